# baseline (speedup 1.0000x reference)
import functools
from functools import partial

import jax
import jax.numpy as jnp
from jax import lax
from jax.experimental import pallas as pl
from jax.experimental.pallas import tpu as pltpu
from jax.experimental.pallas import tpu_sc as plsc

N_NODES = 10000
N_EDGES = 320000
EDGE_CHUNK = 64
NS = 16
NC = 2
ROWS_TOTAL = 5120
E_PAD = ROWS_TOTAL * EDGE_CHUNK
N_PAD = 10112
ROWS_OUT_PER_TILE = N_PAD // NS
IDX_BLOCK = 32
NRING = 4


def _sc_agg_body(split_edges, ib, hstk, src3, dst2, zrows, out, rb0, rb1, rb2,
                 rb3, sb0, sb1, db0, db1, agg, g0, g1, g2, g3, s0, s1, s2,
                 s3, isem_s, isem_d):
    c = lax.axis_index("c")
    s = lax.axis_index("s")
    rbufs = [rb0, rb1, rb2, rb3]
    sbufs = [sb0, sb1]
    dbufs = [db0, db1]
    gsem = [g0, g1, g2, g3]
    ssem = [s0, s1, s2, s3]
    if split_edges:
        rows = ROWS_TOTAL // (NC * NS)
        idx_base = (c * NS + s) * rows
        src_base = idx_base
    else:
        rows = ROWS_TOTAL // NS
        idx_base = s * rows
        src_base = c * ROWS_TOTAL + idx_base
    nblocks = rows // ib

    r0 = s * ROWS_OUT_PER_TILE
    pltpu.sync_copy(zrows, agg.at[pl.ds(r0, ROWS_OUT_PER_TILE)])
    pltpu.async_copy(src3.at[pl.ds(src_base, ib)], sbufs[0], isem_s)
    pltpu.async_copy(dst2.at[pl.ds(idx_base, ib)], dbufs[0], isem_d)
    plsc.subcore_barrier()

    for b in range(nblocks):
        S, D = sbufs[b % 2], dbufs[b % 2]
        pltpu.make_async_copy(src3.at[pl.ds(src_base, ib)], S, isem_s).wait()
        pltpu.make_async_copy(dst2.at[pl.ds(idx_base, ib)], D, isem_d).wait()

        for i in range(NRING):
            if b > 0:
                pltpu.make_async_copy(rbufs[i], agg.at[pl.ds(0, EDGE_CHUNK)],
                                      ssem[i]).wait()
            if i < NRING - 1:
                pltpu.async_copy(hstk.at[S.at[i]], rbufs[i], gsem[i])

        if b + 1 < nblocks:
            pltpu.async_copy(src3.at[pl.ds(src_base + (b + 1) * ib, ib)],
                             sbufs[(b + 1) % 2], isem_s)
            pltpu.async_copy(dst2.at[pl.ds(idx_base + (b + 1) * ib, ib)],
                             dbufs[(b + 1) % 2], isem_d)

        @pl.loop(0, ib, step=NRING)
        def _(j):
            for i in range(NRING):
                k = j + i
                la = (i + 3) % 4

                @pl.when(k + 3 < ib)
                def _():
                    @pl.when(k > 0)
                    def _():
                        pltpu.make_async_copy(
                            rbufs[la], agg.at[pl.ds(0, EDGE_CHUNK)],
                            ssem[la]).wait()
                    pltpu.async_copy(hstk.at[S.at[k + 3]], rbufs[la],
                                     gsem[la])

                pltpu.make_async_copy(hstk.at[S.at[k]], rbufs[i],
                                      gsem[i]).wait()
                pltpu.async_copy(rbufs[i], agg.at[D.at[k]], ssem[i],
                                 add=True)

    for i in range(NRING):
        pltpu.make_async_copy(rbufs[i], agg.at[pl.ds(0, EDGE_CHUNK)],
                              ssem[i]).wait()

    plsc.subcore_barrier()

    pltpu.sync_copy(agg.at[pl.ds(r0, ROWS_OUT_PER_TILE)],
                    out.at[pl.ds(c * N_PAD + r0, ROWS_OUT_PER_TILE)])


@functools.lru_cache(maxsize=None)
def _make_sc_agg(split_edges):
    mesh = plsc.VectorSubcoreMesh(core_axis_name="c", subcore_axis_name="s")
    ib = IDX_BLOCK
    return pl.kernel(
        partial(_sc_agg_body, split_edges, ib),
        out_type=jax.ShapeDtypeStruct((NC * N_PAD, 128), jnp.float32),
        mesh=mesh,
        scratch_types=(
            [pltpu.VMEM((EDGE_CHUNK, 128), jnp.float32)] * NRING +
            [pltpu.VMEM((ib, EDGE_CHUNK), jnp.int32)] * 2 +
            [pltpu.VMEM((ib, EDGE_CHUNK), jnp.int32)] * 2 +
            [pltpu.VMEM_SHARED((N_PAD, 128), jnp.float32)] +
            [pltpu.SemaphoreType.DMA] * (2 * NRING + 2)
        ),
    )


def _bn_relu(h, g, b):
    mu = jnp.mean(h, axis=0)
    var = jnp.mean(h * h, axis=0) - mu * mu
    return jnp.maximum((h - mu) * (g * lax.rsqrt(var + 1e-5)) + b, 0.0)


def _tc_mlp_body(stacked_in, final, hstk_ref, agg_ref, W1_ref, b1_ref, g1_ref,
                 be1_ref, W2_ref, b2_ref, g2_ref, be2_ref, og_ref, ob_ref,
                 out_ref):
    if stacked_in:
        h = jnp.concatenate(
            [hstk_ref[:N_NODES, :], hstk_ref[N_NODES:, :]], axis=1)
        agg = jnp.concatenate(
            [agg_ref[:N_NODES, :], agg_ref[N_PAD:N_PAD + N_NODES, :]], axis=1)
    else:
        h = hstk_ref[...]
        agg = agg_ref[:N_NODES, :] + agg_ref[N_PAD:N_PAD + N_NODES, :]
    h = h + agg
    h = jnp.dot(h, W1_ref[...], preferred_element_type=jnp.float32) + b1_ref[...]
    h = _bn_relu(h, g1_ref[...], be1_ref[...])
    h = jnp.dot(h, W2_ref[...], preferred_element_type=jnp.float32) + b2_ref[...]
    h = _bn_relu(h, g2_ref[...], be2_ref[...])
    h = _bn_relu(h, og_ref[...], ob_ref[...])
    if final:
        m = jnp.max(h, axis=1, keepdims=True)
        e = h - m
        lse = jnp.log(jnp.sum(jnp.exp(e), axis=1, keepdims=True))
        out_ref[...] = e - lse
    else:
        D2 = h.shape[1] // 2
        out_ref[:N_NODES, :] = h[:, :D2]
        out_ref[N_NODES:, :] = h[:, D2:]


@functools.lru_cache(maxsize=None)
def _make_tc_mlp(stacked_in, C_out, final):
    if final:
        oshape = jax.ShapeDtypeStruct((N_NODES, C_out), jnp.float32)
    else:
        oshape = jax.ShapeDtypeStruct((2 * N_NODES, C_out // 2), jnp.float32)
    return pl.pallas_call(
        partial(_tc_mlp_body, stacked_in, final),
        out_shape=oshape,
    )


def kernel(x, edge_index, batch, params):
    src = edge_index[0].astype(jnp.int32)
    dst = edge_index[1].astype(jnp.int32)
    pad = E_PAD - N_EDGES
    src_p = jnp.concatenate([src, jnp.zeros((pad,), jnp.int32)])
    dst_p = jnp.concatenate([dst, jnp.full((pad,), N_NODES, jnp.int32)])
    src2 = src_p.reshape(ROWS_TOTAL, EDGE_CHUNK)
    src3 = jnp.concatenate([src2, src2 + N_NODES], axis=0)
    dst2 = dst_p.reshape(ROWS_TOTAL, EDGE_CHUNK)
    zrows = jnp.zeros((ROWS_OUT_PER_TILE, 128), jnp.float32)

    h = x
    n_layers = len(params)
    for i, p in enumerate(params):
        split_edges = i == 0
        agg = _make_sc_agg(split_edges)(h, src3, dst2, zrows)
        C_out = p['W1'].shape[1]
        final = i == n_layers - 1
        h = _make_tc_mlp(not split_edges, C_out, final)(
            h, agg, p['W1'], p['b1'], p['g1'], p['be1'], p['W2'], p['b2'],
            p['g2'], p['be2'], p['og'], p['ob'])
    return h
